# Initial kernel scaffold; baseline (speedup 1.0000x reference)
import jax
import jax.numpy as jnp
from jax.experimental import pallas as pl


def kernel(x, edge_index, edge_attr, batch, text_embeddings, params):
    raise NotImplementedError("write your pallas kernel here")


# speedup vs baseline: 5.0880x; 5.0880x over previous
import functools

import jax
import jax.numpy as jnp
from jax import lax
from jax.experimental import pallas as pl
from jax.experimental.pallas import tpu as pltpu
from jax.experimental.pallas import tpu_sc as plsc

N = 10000
E = 320000
F32 = jnp.float32


def _leaky(v):
    return jnp.where(v >= 0, v, 0.01 * v)


def _dg_nt(a, w):
    return lax.dot_general(a, w, (((1,), (1,)), ((), ())),
                           precision=lax.Precision.HIGHEST,
                           preferred_element_type=F32)


def _dot(a, b, dims):
    return lax.dot_general(a, b, (dims, ((), ())),
                           precision=lax.Precision.HIGHEST,
                           preferred_element_type=F32)


def _rowln(h, w, b):
    m = jnp.mean(h, axis=-1, keepdims=True)
    v = jnp.mean((h - m) ** 2, axis=-1, keepdims=True)
    return (h - m) * lax.rsqrt(v + 1e-5) * w + b


def _graphln(h, w, b):
    m = jnp.mean(h)
    v = jnp.mean((h - m) ** 2)
    return (h - m) * lax.rsqrt(v + 1e-5) * w + b


def _node_enc_body(x_ref, xeW, xeb, lnw, lnb, Wl, bl, Wr, br, xlr_o):
    h = _dg_nt(x_ref[...], xeW[...]) + xeb[...]
    h = _leaky(_graphln(h, lnw[...], lnb[...]))
    xlr_o[...] = jnp.concatenate(
        [_dg_nt(h, Wl[...]) + bl[...], _dg_nt(h, Wr[...]) + br[...]], axis=1)


_BE = 4000


def _edge_enc_body(ea_ref, eeW, eeb, lnw, lnb, out_ref):
    a = _dg_nt(ea_ref[...], eeW[...]) + eeb[...]
    out_ref[...] = _leaky(_rowln(a, lnw[...], lnb[...]))


def _vperm(vec, idx):
    dn = lax.GatherDimensionNumbers(offset_dims=(), collapsed_slice_dims=(0,),
                                    start_index_map=(0,))
    return lax.gather(vec, idx.reshape(16, 1), dn, (1,),
                      mode=lax.GatherScatterMode.PROMISE_IN_BOUNDS)


def _bcast16(vec, j):
    return _vperm(vec, jnp.full((16,), j, jnp.int32))


def _hsum16(vec, lanes):
    for k in (8, 4, 2, 1):
        vec = vec + _vperm(vec, (lanes + k) & 15)
    return vec


def _gat_sc_call(xlr, src, dst, eaf, we0, we1, att):
    info = plsc.get_sparse_core_info()
    NC, NS, L = info.num_cores, info.num_subcores, info.num_lanes
    NW = NC * NS
    EW = E // NW
    C = 80
    NCHUNK = EW // C
    G = C // 16
    NP = 10240
    RT = NP // NS
    RC = 128
    mesh = plsc.VectorSubcoreMesh(core_axis_name="c", subcore_axis_name="s")

    @functools.partial(
        pl.kernel, mesh=mesh,
        out_type=jax.ShapeDtypeStruct((NC, NP, 128), F32),
        scratch_types=[
            pltpu.VMEM((C,), jnp.int32),
            pltpu.VMEM((C,), jnp.int32),
            pltpu.VMEM((2 * C,), F32),
            pltpu.VMEM((C, 128), F32),
            pltpu.VMEM((C, 128), F32),
            pltpu.VMEM((C, 128), F32),
            pltpu.VMEM((64,), F32),
            pltpu.VMEM((64,), F32),
            pltpu.VMEM((64,), F32),
            pltpu.VMEM((RC, 128), F32),
            pltpu.VMEM_SHARED((NP, 128), F32),
            pltpu.SemaphoreType.DMA,
            pltpu.SemaphoreType.DMA,
        ])
    def gat(xlr_h, src_h, dst_h, ea_h, we0_h, we1_h, att_h, out_h,
            srcv, dstv, eav, xsv, xdv, contv, attv, we0v, we1v,
            obuf, acc, s1, s2):
        cid = lax.axis_index("c")
        sid = lax.axis_index("s")
        wid = sid * NC + cid
        lanes = lax.iota(jnp.int32, L)
        zeros_i = jnp.zeros((L,), jnp.int32)
        zf = jnp.zeros((L,), F32)

        pltpu.sync_copy(att_h, attv)
        pltpu.sync_copy(we0_h, we0v)
        pltpu.sync_copy(we1_h, we1v)

        def zrow(r, carry):
            for kk in range(8):
                obuf[r, pl.ds(kk * 16, 16)] = zf
            return carry
        lax.fori_loop(0, RC, zrow, 0)
        def zcont(r, carry):
            for kk in range(8):
                contv[r, pl.ds(kk * 16, 16)] = zf
            return carry
        lax.fori_loop(0, C, zcont, 0)

        def zacc(t, carry):
            pltpu.sync_copy(obuf, acc.at[pl.ds(sid * RT + t * RC, RC)])
            return carry
        lax.fori_loop(0, RT // RC, zacc, 0)
        plsc.subcore_barrier()

        def chunk(i, carry):
            base = wid * EW + i * C
            pltpu.sync_copy(src_h.at[pl.ds(base, C)], srcv)
            pltpu.sync_copy(dst_h.at[pl.ds(base, C)], dstv)
            pltpu.sync_copy(ea_h.at[pl.ds(2 * base, 2 * C)], eav)
            h1 = pltpu.async_copy(xlr_h.at[srcv], xsv, s1)
            h2 = pltpu.async_copy(xlr_h.at[dstv], xdv, s2)
            h1.wait()
            h2.wait()

            def group(g, gcarry):
                e0 = g * 16
                eaw0 = eav[pl.ds(2 * e0, 16)]
                eaw1 = eav[pl.ds(2 * e0 + 16, 16)]
                logits = zf
                for j in range(16):
                    e = e0 + j
                    eaw = eaw0 if j < 8 else eaw1
                    ea0 = _bcast16(eaw, (2 * j) % 16)
                    ea1 = _bcast16(eaw, (2 * j + 1) % 16)
                    accv = zf
                    for kk in range(4):
                        sl = pl.ds(kk * 16, 16)
                        sr = pl.ds(64 + kk * 16, 16)
                        ev = (xsv[e, sl] + xdv[e, sr]
                              + ea0 * we0v[sl] + ea1 * we1v[sl])
                        ev = jnp.where(ev >= 0, ev, 0.01 * ev)
                        accv = accv + ev * attv[sl]
                    logits = jnp.where(lanes == j, _hsum16(accv, lanes), logits)
                al = jnp.exp(logits)
                for j in range(16):
                    e = e0 + j
                    alb = _bcast16(al, j)
                    for kk in range(4):
                        sl = pl.ds(kk * 16, 16)
                        contv[e, sl] = xsv[e, sl] * alb
                    contv[e, pl.ds(64, 16)] = jnp.where(lanes == 0, alb, 0.0)
                return gcarry
            lax.fori_loop(0, G, group, 0)
            pltpu.sync_copy(contv, acc.at[dstv], add=True)
            return carry
        lax.fori_loop(0, NCHUNK, chunk, 0)
        plsc.subcore_barrier()

        def cpout(t, carry):
            r0 = sid * RT + t * RC
            pltpu.sync_copy(acc.at[pl.ds(r0, RC)], obuf)
            pltpu.sync_copy(obuf, out_h.at[cid, pl.ds(r0, RC)])
            return carry
        lax.fori_loop(0, RT // RC, cpout, 0)

    return gat(xlr, src, dst, eaf, we0, we1, att)


def _post_gat_proj_body(acc_ref, bias, lnw, lnb, Wl, bl, Wr, br, xlr_o):
    a = (acc_ref[0] + acc_ref[1])[:N]
    o = a[:, :64] / (a[:, 64:65] + 1e-16) + bias[...]
    o = _leaky(_graphln(o, lnw[...], lnb[...]))
    xlr_o[...] = jnp.concatenate(
        [_dg_nt(o, Wl[...]) + bl[...], _dg_nt(o, Wr[...]) + br[...]], axis=1)


def _post_gat_h_body(acc_ref, bias, lnw, lnb, h_o):
    a = (acc_ref[0] + acc_ref[1])[:N]
    o = a[:, :64] / (a[:, 64:65] + 1e-16) + bias[...]
    h_o[...] = _leaky(_graphln(o, lnw[...], lnb[...]))


_BN = 1000


def _ca_pool_body(h_ref, batch_ref, text_ref, *refs):
    ca_refs = refs[:28]
    d1W, d1b, lnfw, lnfb, outW, outb, out_ref, sums_s, cnt_s = refs[28:]
    i = pl.program_id(0)
    h = h_ref[...]
    text = text_ref[...]
    for c in range(2):
        (qw, qb, kw, kb, vw, vb, ow, ob,
         l1w, l1b, dW, db, l2w, l2b) = ca_refs[c * 14:(c + 1) * 14]
        q = _dg_nt(h, qw[...]) + qb[...]
        k = _dg_nt(text, kw[...]) + kb[...]
        v = _dg_nt(text, vw[...]) + vb[...]
        ohs = []
        for hh in range(16):
            s = slice(hh * 4, hh * 4 + 4)
            sc = _dot(q[:, s], k[:, s], ((1,), (1,))) * 0.5
            mx = jnp.max(sc, axis=-1, keepdims=True)
            ex = jnp.exp(sc - mx)
            pr = ex / jnp.sum(ex, axis=-1, keepdims=True)
            ohs.append(_dot(pr, v[:, s], ((1,), (0,))))
        o = jnp.concatenate(ohs, axis=1)
        a = _dg_nt(o, ow[...]) + ob[...]
        hq = _rowln(a + h, l1w[...], l1b[...])
        h2 = _dg_nt(hq, dW[...]) + db[...] + hq
        h = _rowln(h2, l2w[...], l2b[...])
    oh = (batch_ref[...] == lax.broadcasted_iota(jnp.int32, (_BN, 16), 1)
          ).astype(F32)
    bs = _dot(oh, h, ((0,), (0,)))
    bc = _dot(oh, jnp.ones((_BN, 1), F32), ((0,), (0,)))

    @pl.when(i == 0)
    def _init():
        sums_s[...] = jnp.zeros_like(sums_s)
        cnt_s[...] = jnp.zeros_like(cnt_s)

    sums_s[...] += bs
    cnt_s[...] += bc

    @pl.when(i == (N // _BN) - 1)
    def _final():
        g = sums_s[...] / jnp.maximum(cnt_s[...], 1.0)
        g = _leaky(_rowln(_dg_nt(g, d1W[...]) + d1b[...],
                          lnfw[...], lnfb[...]))
        out_ref[...] = _dg_nt(g, outW[...]) + outb[...]


def _full(shape):
    return pl.BlockSpec(shape, lambda i: tuple(0 for _ in shape))


def kernel(x, edge_index, edge_attr, batch, text_embeddings, params):
    p = params
    src = edge_index[0].astype(jnp.int32)
    dst = edge_index[1].astype(jnp.int32)
    r2 = lambda a: a.reshape(1, -1)

    xlr1 = pl.pallas_call(
        _node_enc_body,
        out_shape=jax.ShapeDtypeStruct((N, 128), F32),
    )(x, p['xeW'], r2(p['xeb']), r2(p['xelnw']), r2(p['xelnb']),
      p['c1Wl'], r2(p['c1bl']), p['c1Wr'], r2(p['c1br']))

    eaf = pl.pallas_call(
        _edge_enc_body,
        grid=(E // _BE,),
        in_specs=[pl.BlockSpec((_BE, 16), lambda i: (i, 0)),
                  _full((2, 16)), _full((1, 2)), _full((1, 2)),
                  _full((1, 2))],
        out_specs=pl.BlockSpec((_BE, 2), lambda i: (i, 0)),
        out_shape=jax.ShapeDtypeStruct((E, 2), F32),
    )(edge_attr, p['eeW'], r2(p['eeb']), r2(p['eelnw']), r2(p['eelnb']))

    eaflat = eaf.reshape(-1)
    acc1 = _gat_sc_call(xlr1, src, dst, eaflat,
                        p['c1We'][:, 0] + 0.0,
                        p['c1We'][:, 1] + 0.0, p['c1att'])

    xlr2 = pl.pallas_call(
        _post_gat_proj_body,
        out_shape=jax.ShapeDtypeStruct((N, 128), F32),
    )(acc1, r2(p['c1bias']), r2(p['ln1w']), r2(p['ln1b']),
      p['c2Wl'], r2(p['c2bl']), p['c2Wr'], r2(p['c2br']))

    acc2 = _gat_sc_call(xlr2, src, dst, eaflat,
                        p['c2We'][:, 0] + 0.0,
                        p['c2We'][:, 1] + 0.0, p['c2att'])

    h3 = pl.pallas_call(
        _post_gat_h_body,
        out_shape=jax.ShapeDtypeStruct((N, 64), F32),
    )(acc2, r2(p['c2bias']), r2(p['ln2w']), r2(p['ln2b']))

    ca_args = []
    for c in ('ca1', 'ca2'):
        Wq, Wk, Wv = jnp.split(p[c + 'inw'], 3, axis=0)
        bq, bk, bv = jnp.split(p[c + 'inb'], 3)
        ca_args += [Wq, r2(bq), Wk, r2(bk), Wv, r2(bv),
                    p[c + 'ow'], r2(p[c + 'ob']),
                    r2(p[c + 'ln1w']), r2(p[c + 'ln1b']),
                    p[c + 'd1W'], r2(p[c + 'd1b']),
                    r2(p[c + 'ln2w']), r2(p[c + 'ln2b'])]

    ca_specs = []
    for a in ca_args:
        ca_specs.append(_full(tuple(a.shape)))

    out = pl.pallas_call(
        _ca_pool_body,
        grid=(N // _BN,),
        in_specs=[pl.BlockSpec((_BN, 64), lambda i: (i, 0)),
                  pl.BlockSpec((_BN, 1), lambda i: (i, 0)),
                  _full((128, 64))] + ca_specs +
                 [_full((256, 64)), _full((1, 256)), _full((1, 256)),
                  _full((1, 256)), _full((18, 256)), _full((1, 18))],
        out_specs=pl.BlockSpec((16, 18), lambda i: (0, 0)),
        out_shape=jax.ShapeDtypeStruct((16, 18), F32),
        scratch_shapes=[pltpu.VMEM((16, 64), F32),
                        pltpu.VMEM((16, 1), F32)],
    )(h3, batch.reshape(-1, 1), text_embeddings, *ca_args,
      p['d1W'], r2(p['d1b']), r2(p['lnfw']), r2(p['lnfb']),
      p['outW'], r2(p['outb']))
    return out


# speedup vs baseline: 7.2994x; 1.4346x over previous
import functools

import jax
import jax.numpy as jnp
from jax import lax
from jax.experimental import pallas as pl
from jax.experimental.pallas import tpu as pltpu
from jax.experimental.pallas import tpu_sc as plsc

N = 10000
E = 320000
F32 = jnp.float32


def _leaky(v):
    return jnp.where(v >= 0, v, 0.01 * v)


def _dg_nt(a, w):
    return lax.dot_general(a, w, (((1,), (1,)), ((), ())),
                           preferred_element_type=F32)


def _dot(a, b, dims):
    return lax.dot_general(a, b, (dims, ((), ())),
                           preferred_element_type=F32)


def _rowln(h, w, b):
    m = jnp.mean(h, axis=-1, keepdims=True)
    v = jnp.mean((h - m) ** 2, axis=-1, keepdims=True)
    return (h - m) * lax.rsqrt(v + 1e-5) * w + b


def _graphln(h, w, b):
    m = jnp.mean(h)
    v = jnp.mean((h - m) ** 2)
    return (h - m) * lax.rsqrt(v + 1e-5) * w + b


def _node_enc_body(x_ref, xeW, xeb, lnw, lnb, Wl, bl, Wr, br, xlr_o):
    h = _dg_nt(x_ref[...], xeW[...]) + xeb[...]
    h = _leaky(_graphln(h, lnw[...], lnb[...]))
    xlr_o[...] = jnp.concatenate(
        [_dg_nt(h, Wl[...]) + bl[...], _dg_nt(h, Wr[...]) + br[...]], axis=1)


_BE = 4000


def _edge_enc_body(ea_ref, eeW, eeb, lnw, lnb, out_ref):
    a = _dg_nt(ea_ref[...], eeW[...]) + eeb[...]
    out_ref[...] = _leaky(_rowln(a, lnw[...], lnb[...]))


def _vperm(vec, idx):
    dn = lax.GatherDimensionNumbers(offset_dims=(), collapsed_slice_dims=(0,),
                                    start_index_map=(0,))
    return lax.gather(vec, idx.reshape(16, 1), dn, (1,),
                      mode=lax.GatherScatterMode.PROMISE_IN_BOUNDS)


def _bcast16(vec, j):
    return _vperm(vec, jnp.full((16,), j, jnp.int32))


def _hsum16(vec, lanes):
    for k in (8, 4, 2, 1):
        vec = vec + _vperm(vec, (lanes + k) & 15)
    return vec


def _gat_sc_call(xlr, src, dst, eaf, we0, we1, att):
    info = plsc.get_sparse_core_info()
    NC, NS, L = info.num_cores, info.num_subcores, info.num_lanes
    NW = NC * NS
    EW = E // NW
    C = 80
    NCHUNK = EW // C
    G = C // 16
    NP = 10240
    RT = NP // NS
    RC = 128
    mesh = plsc.VectorSubcoreMesh(core_axis_name="c", subcore_axis_name="s")

    @functools.partial(
        pl.kernel, mesh=mesh,
        out_type=jax.ShapeDtypeStruct((NC, NP, 128), F32),
        scratch_types=[
            pltpu.VMEM((C,), jnp.int32),
            pltpu.VMEM((C,), jnp.int32),
            pltpu.VMEM((2 * C,), F32),
            pltpu.VMEM((C, 128), F32),
            pltpu.VMEM((C, 128), F32),
            pltpu.VMEM((C, 128), F32),
            pltpu.VMEM((64,), F32),
            pltpu.VMEM((64,), F32),
            pltpu.VMEM((64,), F32),
            pltpu.VMEM((RC, 128), F32),
            pltpu.VMEM_SHARED((NP, 128), F32),
            pltpu.SemaphoreType.DMA,
            pltpu.SemaphoreType.DMA,
        ])
    def gat(xlr_h, src_h, dst_h, ea_h, we0_h, we1_h, att_h, out_h,
            srcv, dstv, eav, xsv, xdv, contv, attv, we0v, we1v,
            obuf, acc, s1, s2):
        cid = lax.axis_index("c")
        sid = lax.axis_index("s")
        wid = sid * NC + cid
        lanes = lax.iota(jnp.int32, L)
        zeros_i = jnp.zeros((L,), jnp.int32)
        zf = jnp.zeros((L,), F32)

        pltpu.sync_copy(att_h, attv)
        pltpu.sync_copy(we0_h, we0v)
        pltpu.sync_copy(we1_h, we1v)

        def zrow(r, carry):
            for kk in range(8):
                obuf[r, pl.ds(kk * 16, 16)] = zf
            return carry
        lax.fori_loop(0, RC, zrow, 0)
        def zcont(r, carry):
            for kk in range(8):
                contv[r, pl.ds(kk * 16, 16)] = zf
            return carry
        lax.fori_loop(0, C, zcont, 0)

        def zacc(t, carry):
            pltpu.sync_copy(obuf, acc.at[pl.ds(sid * RT + t * RC, RC)])
            return carry
        lax.fori_loop(0, RT // RC, zacc, 0)
        plsc.subcore_barrier()

        def chunk(i, carry):
            base = wid * EW + i * C
            pltpu.sync_copy(src_h.at[pl.ds(base, C)], srcv)
            pltpu.sync_copy(dst_h.at[pl.ds(base, C)], dstv)
            pltpu.sync_copy(ea_h.at[pl.ds(2 * base, 2 * C)], eav)
            h1 = pltpu.async_copy(xlr_h.at[srcv], xsv, s1)
            h2 = pltpu.async_copy(xlr_h.at[dstv], xdv, s2)
            h1.wait()
            h2.wait()

            def group(g, gcarry):
                e0 = g * 16
                eaw0 = eav[pl.ds(2 * e0, 16)]
                eaw1 = eav[pl.ds(2 * e0 + 16, 16)]
                logits = zf
                for j in range(16):
                    e = e0 + j
                    eaw = eaw0 if j < 8 else eaw1
                    ea0 = _bcast16(eaw, (2 * j) % 16)
                    ea1 = _bcast16(eaw, (2 * j + 1) % 16)
                    accv = zf
                    for kk in range(4):
                        sl = pl.ds(kk * 16, 16)
                        sr = pl.ds(64 + kk * 16, 16)
                        ev = (xsv[e, sl] + xdv[e, sr]
                              + ea0 * we0v[sl] + ea1 * we1v[sl])
                        ev = jnp.where(ev >= 0, ev, 0.01 * ev)
                        accv = accv + ev * attv[sl]
                    logits = jnp.where(lanes == j, _hsum16(accv, lanes), logits)
                al = jnp.exp(logits)
                for j in range(16):
                    e = e0 + j
                    alb = _bcast16(al, j)
                    for kk in range(4):
                        sl = pl.ds(kk * 16, 16)
                        contv[e, sl] = xsv[e, sl] * alb
                    contv[e, pl.ds(64, 16)] = jnp.where(lanes == 0, alb, 0.0)
                return gcarry
            lax.fori_loop(0, G, group, 0)
            pltpu.sync_copy(contv, acc.at[dstv], add=True)
            return carry
        lax.fori_loop(0, NCHUNK, chunk, 0)
        plsc.subcore_barrier()

        def cpout(t, carry):
            r0 = sid * RT + t * RC
            pltpu.sync_copy(acc.at[pl.ds(r0, RC)], obuf)
            pltpu.sync_copy(obuf, out_h.at[cid, pl.ds(r0, RC)])
            return carry
        lax.fori_loop(0, RT // RC, cpout, 0)

    return gat(xlr, src, dst, eaf, we0, we1, att)


def _post_gat_proj_body(acc_ref, bias, lnw, lnb, Wl, bl, Wr, br, xlr_o):
    a = (acc_ref[0] + acc_ref[1])[:N]
    o = a[:, :64] / (a[:, 64:65] + 1e-16) + bias[...]
    o = _leaky(_graphln(o, lnw[...], lnb[...]))
    xlr_o[...] = jnp.concatenate(
        [_dg_nt(o, Wl[...]) + bl[...], _dg_nt(o, Wr[...]) + br[...]], axis=1)


def _post_gat_h_body(acc_ref, bias, lnw, lnb, h_o):
    a = (acc_ref[0] + acc_ref[1])[:N]
    o = a[:, :64] / (a[:, 64:65] + 1e-16) + bias[...]
    h_o[...] = _leaky(_graphln(o, lnw[...], lnb[...]))


_BN = 1000


def _ca_pool_body(h_ref, batch_ref, text_ref, *refs):
    ca_refs = refs[:28]
    d1W, d1b, lnfw, lnfb, outW, outb, out_ref, sums_s, cnt_s = refs[28:]
    i = pl.program_id(0)
    h = h_ref[...]
    text = text_ref[...]
    for c in range(2):
        (qw, qb, kw, kb, vw, vb, ow, ob,
         l1w, l1b, dW, db, l2w, l2b) = ca_refs[c * 14:(c + 1) * 14]
        q = _dg_nt(h, qw[...]) + qb[...]
        k = _dg_nt(text, kw[...]) + kb[...]
        v = _dg_nt(text, vw[...]) + vb[...]
        ohs = []
        for hh in range(16):
            s = slice(hh * 4, hh * 4 + 4)
            sc = _dot(q[:, s], k[:, s], ((1,), (1,))) * 0.5
            mx = jnp.max(sc, axis=-1, keepdims=True)
            ex = jnp.exp(sc - mx)
            pr = ex / jnp.sum(ex, axis=-1, keepdims=True)
            ohs.append(_dot(pr, v[:, s], ((1,), (0,))))
        o = jnp.concatenate(ohs, axis=1)
        a = _dg_nt(o, ow[...]) + ob[...]
        hq = _rowln(a + h, l1w[...], l1b[...])
        h2 = _dg_nt(hq, dW[...]) + db[...] + hq
        h = _rowln(h2, l2w[...], l2b[...])
    oh = (batch_ref[...] == lax.broadcasted_iota(jnp.int32, (_BN, 16), 1)
          ).astype(F32)
    bs = _dot(oh, h, ((0,), (0,)))
    bc = _dot(oh, jnp.ones((_BN, 1), F32), ((0,), (0,)))

    @pl.when(i == 0)
    def _init():
        sums_s[...] = jnp.zeros_like(sums_s)
        cnt_s[...] = jnp.zeros_like(cnt_s)

    sums_s[...] += bs
    cnt_s[...] += bc

    @pl.when(i == (N // _BN) - 1)
    def _final():
        g = sums_s[...] / jnp.maximum(cnt_s[...], 1.0)
        g = _leaky(_rowln(_dg_nt(g, d1W[...]) + d1b[...],
                          lnfw[...], lnfb[...]))
        out_ref[...] = _dg_nt(g, outW[...]) + outb[...]


def _full(shape):
    return pl.BlockSpec(shape, lambda i: tuple(0 for _ in shape))


def kernel(x, edge_index, edge_attr, batch, text_embeddings, params):
    p = params
    src = edge_index[0].astype(jnp.int32)
    dst = edge_index[1].astype(jnp.int32)
    r2 = lambda a: a.reshape(1, -1)

    xlr1 = pl.pallas_call(
        _node_enc_body,
        out_shape=jax.ShapeDtypeStruct((N, 128), F32),
    )(x, p['xeW'], r2(p['xeb']), r2(p['xelnw']), r2(p['xelnb']),
      p['c1Wl'], r2(p['c1bl']), p['c1Wr'], r2(p['c1br']))

    eaf = pl.pallas_call(
        _edge_enc_body,
        grid=(E // _BE,),
        in_specs=[pl.BlockSpec((_BE, 16), lambda i: (i, 0)),
                  _full((2, 16)), _full((1, 2)), _full((1, 2)),
                  _full((1, 2))],
        out_specs=pl.BlockSpec((_BE, 2), lambda i: (i, 0)),
        out_shape=jax.ShapeDtypeStruct((E, 2), F32),
    )(edge_attr, p['eeW'], r2(p['eeb']), r2(p['eelnw']), r2(p['eelnb']))

    eaflat = eaf.reshape(-1)
    acc1 = _gat_sc_call(xlr1, src, dst, eaflat,
                        p['c1We'][:, 0] + 0.0,
                        p['c1We'][:, 1] + 0.0, p['c1att'])

    xlr2 = pl.pallas_call(
        _post_gat_proj_body,
        out_shape=jax.ShapeDtypeStruct((N, 128), F32),
    )(acc1, r2(p['c1bias']), r2(p['ln1w']), r2(p['ln1b']),
      p['c2Wl'], r2(p['c2bl']), p['c2Wr'], r2(p['c2br']))

    acc2 = _gat_sc_call(xlr2, src, dst, eaflat,
                        p['c2We'][:, 0] + 0.0,
                        p['c2We'][:, 1] + 0.0, p['c2att'])

    h3 = pl.pallas_call(
        _post_gat_h_body,
        out_shape=jax.ShapeDtypeStruct((N, 64), F32),
    )(acc2, r2(p['c2bias']), r2(p['ln2w']), r2(p['ln2b']))

    ca_args = []
    for c in ('ca1', 'ca2'):
        Wq, Wk, Wv = jnp.split(p[c + 'inw'], 3, axis=0)
        bq, bk, bv = jnp.split(p[c + 'inb'], 3)
        ca_args += [Wq, r2(bq), Wk, r2(bk), Wv, r2(bv),
                    p[c + 'ow'], r2(p[c + 'ob']),
                    r2(p[c + 'ln1w']), r2(p[c + 'ln1b']),
                    p[c + 'd1W'], r2(p[c + 'd1b']),
                    r2(p[c + 'ln2w']), r2(p[c + 'ln2b'])]

    ca_specs = []
    for a in ca_args:
        ca_specs.append(_full(tuple(a.shape)))

    out = pl.pallas_call(
        _ca_pool_body,
        grid=(N // _BN,),
        in_specs=[pl.BlockSpec((_BN, 64), lambda i: (i, 0)),
                  pl.BlockSpec((_BN, 1), lambda i: (i, 0)),
                  _full((128, 64))] + ca_specs +
                 [_full((256, 64)), _full((1, 256)), _full((1, 256)),
                  _full((1, 256)), _full((18, 256)), _full((1, 18))],
        out_specs=pl.BlockSpec((16, 18), lambda i: (0, 0)),
        out_shape=jax.ShapeDtypeStruct((16, 18), F32),
        scratch_shapes=[pltpu.VMEM((16, 64), F32),
                        pltpu.VMEM((16, 1), F32)],
    )(h3, batch.reshape(-1, 1), text_embeddings, *ca_args,
      p['d1W'], r2(p['d1b']), r2(p['lnfw']), r2(p['lnfb']),
      p['outW'], r2(p['outb']))
    return out
